# baseline (speedup 1.0000x reference)
import functools

import jax
import jax.numpy as jnp
from jax import lax
from jax.experimental import pallas as pl
from jax.experimental.pallas import tpu as pltpu
from jax.experimental.pallas import tpu_sc as plsc

_B, _L = 4096, 200
_V = 1000
_FORM_D, _LEMMA_D, _TAG_D, _FEATS_D = 64, 64, 32, 32
_OUT_D = 192
_NC, _NS = 2, 16
_NW = _NC * _NS
_BPW = _B // _NW
_LC = 8
_NLR = _L // _LC
_NC8 = _OUT_D // 8


def _sc_body(lat_hbm, form_hbm, lemma_hbm, tag_hbm, feats_hbm, out_hbm,
             form_sh, lemma_sh, tag_sh, feats_sh,
             lat_v, if_v, il_v, it_v, is0_v, is1_v, is2_v, is3_v, is4_v,
             is5_v, form_v, lemma_v, tag_v, f_v, blkA, blkB,
             sem_g, semA, semB):
    sid = lax.axis_index("s")
    wid = sid * _NC + lax.axis_index("c")

    @pl.when(sid == 0)
    def _stage_tables():
        pltpu.sync_copy(form_hbm, form_sh)
        pltpu.sync_copy(lemma_hbm, lemma_sh)
        pltpu.sync_copy(tag_hbm, tag_sh)
        pltpu.sync_copy(feats_hbm, feats_sh)

    plsc.subcore_barrier()

    idx_v = [if_v, il_v, it_v, is0_v, is1_v, is2_v, is3_v, is4_v, is5_v]

    ii = lax.iota(jnp.int32, 16)
    c8v = [(ii + 16 * g) >> 3 for g in range(12)]
    csv = [(ii + 16 * g) & 7 for g in range(12)]

    def extract_idx(li):
        def kblk(k, carry):
            bb = ii + k * 16
            lv = jnp.full((16,), li, jnp.int32)
            for j in range(9):
                jv = jnp.full((16,), j, jnp.int32)
                idx_v[j][pl.ds(k * 16, 16)] = plsc.load_gather(
                    lat_v, [bb, lv, jv])
            return carry

        lax.fori_loop(0, _BPW // 16, kblk, 0)

    def fire_gathers():
        cps = [
            pltpu.async_copy(form_sh.at[if_v], form_v, sem_g),
            pltpu.async_copy(lemma_sh.at[il_v], lemma_v, sem_g),
            pltpu.async_copy(tag_sh.at[it_v], tag_v, sem_g),
        ]
        for j in range(6):
            cps.append(
                pltpu.async_copy(feats_sh.at[idx_v[3 + j]],
                                 f_v.at[pl.ds(j * _BPW, _BPW)], sem_g))
        return cps

    def transpose_into(blk):
        def row(bl, carry):
            blv = jnp.full((16,), bl, jnp.int32)
            for g in range(4):
                x = form_v[bl, pl.ds(16 * g, 16)]
                plsc.store_scatter(blk, [c8v[g], csv[g], blv], x)
            for g in range(4):
                x = lemma_v[bl, pl.ds(16 * g, 16)]
                plsc.store_scatter(blk, [c8v[4 + g], csv[4 + g], blv], x)
            for g in range(2):
                x = tag_v[bl, pl.ds(16 * g, 16)]
                plsc.store_scatter(blk, [c8v[8 + g], csv[8 + g], blv], x)
            for g in range(2):
                h = 16 * g
                acc = f_v[bl, pl.ds(h, 16)]
                for j in range(1, 6):
                    acc = acc + f_v[j * _BPW + bl, pl.ds(h, 16)]
                plsc.store_scatter(blk, [c8v[10 + g], csv[10 + g], blv], acc)
            return carry

        lax.fori_loop(0, _BPW, row, 0, unroll=2)

    def drain_write(blk, sem):
        pltpu.make_async_copy(blk, out_hbm.at[0, pl.ds(0, _NC8), 0],
                              sem).wait()

    def round_body(r, carry):
        l0 = r * _LC
        pltpu.sync_copy(
            lat_hbm.at[pl.ds(wid * _BPW, _BPW), pl.ds(l0, _LC), :], lat_v)

        for li in range(_LC):
            blk, sem = (blkA, semA) if li % 2 == 0 else (blkB, semB)
            extract_idx(li)
            cps = fire_gathers()
            for cp in cps:
                cp.wait()

            not_first = jnp.logical_or(r > 0, li >= 2)

            @pl.when(not_first)
            def _():
                drain_write(blk, sem)

            transpose_into(blk)
            pltpu.async_copy(blk, out_hbm.at[l0 + li, pl.ds(0, _NC8), wid],
                             sem)
        return carry

    lax.fori_loop(0, _NLR, round_body, 0)
    drain_write(blkA, semA)
    drain_write(blkB, semB)


@jax.jit
def _morph_embed(lattice, form_t, lemma_t, tag_t, feats_t):
    mesh = plsc.VectorSubcoreMesh(core_axis_name="c", subcore_axis_name="s")
    kern = functools.partial(
        pl.kernel,
        mesh=mesh,
        out_type=jax.ShapeDtypeStruct((_L, _NC8, _NW, 8, _BPW), jnp.float32),
        scratch_types=(
            [
                pltpu.VMEM_SHARED((_V, _FORM_D), jnp.float32),
                pltpu.VMEM_SHARED((_V, _LEMMA_D), jnp.float32),
                pltpu.VMEM_SHARED((_V, _TAG_D), jnp.float32),
                pltpu.VMEM_SHARED((_V, _FEATS_D), jnp.float32),
                pltpu.VMEM((_BPW, _LC, 9), jnp.int32),
            ]
            + [pltpu.VMEM((_BPW,), jnp.int32)] * 9 + [
                pltpu.VMEM((_BPW, _FORM_D), jnp.float32),
                pltpu.VMEM((_BPW, _LEMMA_D), jnp.float32),
                pltpu.VMEM((_BPW, _TAG_D), jnp.float32),
                pltpu.VMEM((6 * _BPW, _FEATS_D), jnp.float32),
                pltpu.VMEM((_NC8, 8, _BPW), jnp.float32),
                pltpu.VMEM((_NC8, 8, _BPW), jnp.float32),
                pltpu.SemaphoreType.DMA,
                pltpu.SemaphoreType.DMA,
                pltpu.SemaphoreType.DMA,
            ]
        ),
        compiler_params=pltpu.CompilerParams(use_tc_tiling_on_sc=False,
                                             needs_layout_passes=False),
    )(_sc_body)
    return kern(lattice, form_t, lemma_t, tag_t, feats_t)


def kernel(lattice, W_form, W_lemma, W_tag, W_feats):
    a = _morph_embed(lattice, W_form[:_V], W_lemma[:_V], W_tag[:_V],
                     W_feats[:_V] * (1.0 / 6.0))
    return a.transpose(2, 4, 0, 1, 3).reshape(_B, _L, _OUT_D)


# speedup vs baseline: 2.0089x; 2.0089x over previous
import functools

import jax
import jax.numpy as jnp
from jax import lax
from jax.experimental import pallas as pl
from jax.experimental.pallas import tpu as pltpu
from jax.experimental.pallas import tpu_sc as plsc

_B, _L = 4096, 200
_BL = _B * _L
_V = 1000
_FORM_D, _LEMMA_D, _TAG_D, _FEATS_D = 64, 64, 32, 32
_OUT_D = 192

_NC, _NS = 2, 16
_NW = _NC * _NS
_PW = _BL // _NW
_C = 128
_SUPER = 1024
_NCH = _SUPER // _C
_NPAIR = _NCH // 2
_NSUPER = _PW // _SUPER


def _sc_body(lat_hbm, form_hbm, lemma_hbm, tag_hbm, feats_hbm, out_hbm,
             form_sh, lemma_sh, tag_sh, feats_sh,
             lat_v, if_v, il_v, it_v, is0_v, is1_v, is2_v, is3_v, is4_v,
             is5_v,
             formA, lemmaA, tagA, fA, accA,
             formB, lemmaB, tagB, fB, accB,
             sem_i, sem_g, semA, semB):
    sid = lax.axis_index("s")
    wid = sid * _NC + lax.axis_index("c")

    @pl.when(sid == 0)
    def _stage_tables():
        pltpu.sync_copy(form_hbm, form_sh)
        pltpu.sync_copy(lemma_hbm, lemma_sh)
        pltpu.sync_copy(tag_hbm, tag_sh)
        pltpu.sync_copy(feats_hbm, feats_sh)

    plsc.subcore_barrier()

    idx_v = [if_v, il_v, it_v, is0_v, is1_v, is2_v, is3_v, is4_v, is5_v]

    def fire_gathers(off, form_v, lemma_v, tag_v, f_v):
        cps = [
            pltpu.async_copy(form_sh.at[if_v.at[pl.ds(off, _C)]], form_v,
                             sem_g),
            pltpu.async_copy(lemma_sh.at[il_v.at[pl.ds(off, _C)]], lemma_v,
                             sem_g),
            pltpu.async_copy(tag_sh.at[it_v.at[pl.ds(off, _C)]], tag_v,
                             sem_g),
        ]
        for j in range(6):
            cps.append(
                pltpu.async_copy(feats_sh.at[idx_v[3 + j].at[pl.ds(off, _C)]],
                                 f_v.at[pl.ds(j * _C, _C)], sem_g))
        return cps

    def feat_sum(f_v, acc_v):
        def row(p, carry):
            for h in (0, 16):
                a = f_v[p, pl.ds(h, 16)]
                for j in range(1, 6):
                    a = a + f_v[j * _C + p, pl.ds(h, 16)]
                acc_v[p, pl.ds(h, 16)] = a
            return carry

        lax.fori_loop(0, _C, row, 0, unroll=2)

    def fire_writes(base, form_v, lemma_v, tag_v, acc_v, sem):
        r = pl.ds(base, _C)
        pltpu.async_copy(form_v, out_hbm.at[r, pl.ds(0, _FORM_D)], sem)
        pltpu.async_copy(lemma_v, out_hbm.at[r, pl.ds(_FORM_D, _LEMMA_D)],
                         sem)
        pltpu.async_copy(tag_v, out_hbm.at[r, pl.ds(128, _TAG_D)], sem)
        pltpu.async_copy(acc_v, out_hbm.at[r, pl.ds(160, _FEATS_D)], sem)

    def drain_writes(form_v, lemma_v, tag_v, acc_v, sem):
        r = pl.ds(0, _C)
        pltpu.make_async_copy(form_v, out_hbm.at[r, pl.ds(0, _FORM_D)],
                              sem).wait()
        pltpu.make_async_copy(lemma_v,
                              out_hbm.at[r, pl.ds(_FORM_D, _LEMMA_D)],
                              sem).wait()
        pltpu.make_async_copy(tag_v, out_hbm.at[r, pl.ds(128, _TAG_D)],
                              sem).wait()
        pltpu.make_async_copy(acc_v, out_hbm.at[r, pl.ds(160, _FEATS_D)],
                              sem).wait()

    bufsA = (formA, lemmaA, tagA, fA, accA)
    bufsB = (formB, lemmaB, tagB, fB, accB)

    def half(off, bufs, semW, not_first):
        form_v, lemma_v, tag_v, f_v, acc_v = bufs

        @pl.when(not_first)
        def _():
            drain_writes(form_v, lemma_v, tag_v, acc_v, semW)

        return fire_gathers(off, form_v, lemma_v, tag_v, f_v)

    def finish(base, bufs, semW, cps):
        form_v, lemma_v, tag_v, f_v, acc_v = bufs
        for cp in cps:
            cp.wait()
        feat_sum(f_v, acc_v)
        fire_writes(base, form_v, lemma_v, tag_v, acc_v, semW)

    def super_body(s, carry):
        sbase = wid * _PW + s * _SUPER

        pltpu.sync_copy(lat_hbm.at[pl.ds(sbase * 9, _SUPER * 9)], lat_v)

        def kblk(k, carry2):
            bvec = lax.iota(jnp.int32, 16) * 9 + k * 144
            for j in range(9):
                idx_v[j][pl.ds(k * 16, 16)] = plsc.load_gather(
                    lat_v, [bvec + j])
            return carry2

        lax.fori_loop(0, _SUPER // 16, kblk, 0)

        def pair_body(p, carry2):
            a_off = (2 * p) * _C
            a_base = sbase + a_off
            b_base = a_base + _C
            not_first = jnp.logical_or(s > 0, p > 0)

            ga = half(a_off, bufsA, semA, not_first)
            gb = half(a_off + _C, bufsB, semB, not_first)
            finish(a_base, bufsA, semA, ga)
            finish(b_base, bufsB, semB, gb)
            return carry2

        lax.fori_loop(0, _NPAIR, pair_body, 0)
        return carry

    lax.fori_loop(0, _NSUPER, super_body, 0)
    drain_writes(formA, lemmaA, tagA, accA, semA)
    drain_writes(formB, lemmaB, tagB, accB, semB)


@jax.jit
def _morph_embed(lat_flat, form_t, lemma_t, tag_t, feats_t):
    mesh = plsc.VectorSubcoreMesh(core_axis_name="c", subcore_axis_name="s")
    ring = [
        pltpu.VMEM((_C, _FORM_D), jnp.float32),
        pltpu.VMEM((_C, _LEMMA_D), jnp.float32),
        pltpu.VMEM((_C, _TAG_D), jnp.float32),
        pltpu.VMEM((6 * _C, _FEATS_D), jnp.float32),
        pltpu.VMEM((_C, _FEATS_D), jnp.float32),
    ]
    kern = functools.partial(
        pl.kernel,
        mesh=mesh,
        out_type=jax.ShapeDtypeStruct((_BL, _OUT_D), jnp.float32),
        scratch_types=(
            [
                pltpu.VMEM_SHARED((_V, _FORM_D), jnp.float32),
                pltpu.VMEM_SHARED((_V, _LEMMA_D), jnp.float32),
                pltpu.VMEM_SHARED((_V, _TAG_D), jnp.float32),
                pltpu.VMEM_SHARED((_V, _FEATS_D), jnp.float32),
                pltpu.VMEM((_SUPER * 9,), jnp.int32),
            ]
            + [pltpu.VMEM((_SUPER,), jnp.int32)] * 9 + ring + ring + [
                pltpu.SemaphoreType.DMA,
                pltpu.SemaphoreType.DMA,
                pltpu.SemaphoreType.DMA,
                pltpu.SemaphoreType.DMA,
            ]
        ),
        compiler_params=pltpu.CompilerParams(use_tc_tiling_on_sc=False,
                                             needs_layout_passes=False),
    )(_sc_body)
    return kern(lat_flat, form_t, lemma_t, tag_t, feats_t)


def kernel(lattice, W_form, W_lemma, W_tag, W_feats):
    out = _morph_embed(lattice.reshape(-1), W_form[:_V], W_lemma[:_V],
                       W_tag[:_V], W_feats[:_V] * (1.0 / 6.0))
    return out.reshape(_B, _L, _OUT_D)
